# baseline (speedup 1.0000x reference)
import functools

import jax
import jax.numpy as jnp
from jax import lax
from jax.experimental import pallas as pl
from jax.experimental.pallas import tpu as pltpu
from jax.experimental.pallas import tpu_sc as plsc

BATCH = 1024
EMBED_DIM = 32


def _make_sc_gather(V, D, B):
    info = plsc.get_sparse_core_info()
    NC, NS = info.num_cores, info.num_subcores
    NW = NC * NS
    b_per_w = B // NW
    mesh = plsc.VectorSubcoreMesh(core_axis_name="c", subcore_axis_name="s")

    @functools.partial(
        pl.kernel,
        mesh=mesh,
        compiler_params=pltpu.CompilerParams(use_tc_tiling_on_sc=False),
        out_type=jax.ShapeDtypeStruct((B, D), jnp.float32),
        scratch_types=[
            pltpu.VMEM((b_per_w,), jnp.int32),
            pltpu.VMEM((b_per_w, D), jnp.float32),
            pltpu.SemaphoreType.DMA,
        ],
    )
    def sc_gather(table_hbm, idx_hbm, out_hbm, idx_v, rows_v, sem):
        wid = lax.axis_index("s") * NC + lax.axis_index("c")
        base = wid * b_per_w
        pltpu.sync_copy(idx_hbm.at[pl.ds(base, b_per_w)], idx_v)
        pltpu.async_copy(table_hbm.at[idx_v], rows_v, sem).wait()
        pltpu.sync_copy(rows_v, out_hbm.at[pl.ds(base, b_per_w)])

    return sc_gather


def _tc_project_deep(x, WT, b2d, bblk, nbuf):
    B, D = x.shape
    V = WT.shape[1]
    nsteps = B // bblk

    def body(x_ref, wt_ref, b_ref, o_hbm, obuf, sems):
        i = pl.program_id(0)
        slot = lax.rem(i, nbuf)
        row = slot * bblk

        @pl.when(i >= nbuf)
        def _():
            pltpu.make_async_copy(
                obuf.at[pl.ds(row, bblk)],
                o_hbm.at[pl.ds((i - nbuf) * bblk, bblk)],
                sems.at[slot],
            ).wait()

        obuf[pl.ds(row, bblk)] = (
            jnp.dot(x_ref[...], wt_ref[...], preferred_element_type=jnp.float32)
            + b_ref[...]
        )
        copy = pltpu.make_async_copy(
            obuf.at[pl.ds(row, bblk)],
            o_hbm.at[pl.ds(i * bblk, bblk)],
            sems.at[slot],
        )

        @pl.when(lax.rem(i, 2) == 0)
        def _():
            copy.start(priority=0)

        @pl.when(lax.rem(i, 2) == 1)
        def _():
            copy.start(priority=1)

        @pl.when(i == nsteps - 1)
        def _():
            for k in range(nbuf):
                j = i - k

                @pl.when(j >= 0)
                def _():
                    s = lax.rem(j, nbuf)
                    pltpu.make_async_copy(
                        obuf.at[pl.ds(s * bblk, bblk)],
                        o_hbm.at[pl.ds(j * bblk, bblk)],
                        sems.at[s],
                    ).wait()

    return pl.pallas_call(
        body,
        grid=(nsteps,),
        in_specs=[
            pl.BlockSpec((bblk, D), lambda i: (i, 0)),
            pl.BlockSpec((D, V), lambda i: (0, 0)),
            pl.BlockSpec((1, V), lambda i: (0, 0)),
        ],
        out_specs=pl.BlockSpec(memory_space=pltpu.MemorySpace.HBM),
        out_shape=jax.ShapeDtypeStruct((B, V), jnp.float32),
        scratch_shapes=[
            pltpu.VMEM((nbuf * bblk, V), jnp.float32),
            pltpu.SemaphoreType.DMA((nbuf,)),
        ],
    )(x, WT, b2d)


def _bmajor_body(x_ref, wt_ref, b_ref, o_ref):
    o_ref[...] = (
        jnp.dot(x_ref[...], wt_ref[...], preferred_element_type=jnp.float32)
        + b_ref[...]
    )


def _tc_project_bmajor(x, WT, b2d, bblk):
    B, D = x.shape
    V = WT.shape[1]
    return pl.pallas_call(
        _bmajor_body,
        grid=(B // bblk,),
        in_specs=[
            pl.BlockSpec((bblk, D), lambda i: (i, 0)),
            pl.BlockSpec((D, V), lambda i: (0, 0)),
            pl.BlockSpec((1, V), lambda i: (0, 0)),
        ],
        out_specs=pl.BlockSpec((bblk, V), lambda i: (i, 0)),
        out_shape=jax.ShapeDtypeStruct((B, V), jnp.float32),
    )(x, WT, b2d)


_NBUF = 4


def _tc_project(x, W, b2d, nv):
    B, D = x.shape
    V = W.shape[0]
    nsteps = pl.cdiv(V, nv)
    tail = V - (nsteps - 1) * nv

    def body(x_ref, w_ref, b_ref, o_hbm, obuf, tbuf, sems, tsem):
        i = pl.program_id(0)
        slot = lax.rem(i, _NBUF)

        @pl.when(jnp.logical_and(i >= _NBUF, i - _NBUF < nsteps - 1))
        def _():
            pltpu.make_async_copy(
                obuf.at[slot],
                o_hbm.at[:, pl.ds((i - _NBUF) * nv, nv)],
                sems.at[slot],
            ).wait()

        acc = lax.dot_general(
            x_ref[...],
            w_ref[...],
            dimension_numbers=(((1,), (1,)), ((), ())),
            preferred_element_type=jnp.float32,
        )

        @pl.when(i < nsteps - 1)
        def _():
            obuf[slot] = acc + b_ref[...]
            pltpu.make_async_copy(
                obuf.at[slot],
                o_hbm.at[:, pl.ds(i * nv, nv)],
                sems.at[slot],
            ).start()

        @pl.when(i == nsteps - 1)
        def _():
            tbuf[...] = (acc + b_ref[...])[:, :tail]
            pltpu.make_async_copy(
                tbuf,
                o_hbm.at[:, pl.ds((nsteps - 1) * nv, tail)],
                tsem,
            ).start()
            pltpu.make_async_copy(
                tbuf,
                o_hbm.at[:, pl.ds((nsteps - 1) * nv, tail)],
                tsem,
            ).wait()
            for k in range(1, _NBUF):
                j = i - k

                @pl.when(j >= 0)
                def _():
                    s = lax.rem(j, _NBUF)
                    pltpu.make_async_copy(
                        obuf.at[s],
                        o_hbm.at[:, pl.ds(j * nv, nv)],
                        sems.at[s],
                    ).wait()

    return pl.pallas_call(
        body,
        grid=(nsteps,),
        in_specs=[
            pl.BlockSpec((B, D), lambda i: (0, 0)),
            pl.BlockSpec((nv, D), lambda i: (i, 0)),
            pl.BlockSpec((1, nv), lambda i: (0, i)),
        ],
        out_specs=pl.BlockSpec(memory_space=pltpu.MemorySpace.HBM),
        out_shape=jax.ShapeDtypeStruct((B, V), jnp.float32),
        scratch_shapes=[
            pltpu.VMEM((_NBUF, B, nv), jnp.float32),
            pltpu.VMEM((B, tail), jnp.float32),
            pltpu.SemaphoreType.DMA((_NBUF,)),
            pltpu.SemaphoreType.DMA,
        ],
    )(x, W, b2d)


def kernel(inputs, table, W, b):
    V, D = table.shape
    B = inputs.shape[0]
    idx = inputs.astype(jnp.int32)
    x = jnp.take(table, idx, axis=0)
    logits = _tc_project_deep(x, W.T, b.reshape(1, V), 8, 8)
    return logits


# speedup vs baseline: 2.7366x; 2.7366x over previous
import functools

import jax
import jax.numpy as jnp
from jax import lax
from jax.experimental import pallas as pl
from jax.experimental.pallas import tpu as pltpu
from jax.experimental.pallas import tpu_sc as plsc


def _make_sc_gather(V, D, B):
    info = plsc.get_sparse_core_info()
    NC, NS = info.num_cores, info.num_subcores
    NW = NC * NS
    b_per_w = B // NW
    mesh = plsc.VectorSubcoreMesh(core_axis_name="c", subcore_axis_name="s")

    @functools.partial(
        pl.kernel,
        mesh=mesh,
        compiler_params=pltpu.CompilerParams(use_tc_tiling_on_sc=False),
        out_type=jax.ShapeDtypeStruct((B, D), jnp.float32),
        scratch_types=[
            pltpu.VMEM((b_per_w,), jnp.int32),
            pltpu.VMEM((b_per_w, D), jnp.float32),
            pltpu.SemaphoreType.DMA,
        ],
    )
    def sc_gather(table_hbm, idx_hbm, out_hbm, idx_v, rows_v, sem):
        wid = lax.axis_index("s") * NC + lax.axis_index("c")
        base = wid * b_per_w
        pltpu.sync_copy(idx_hbm.at[pl.ds(base, b_per_w)], idx_v)
        pltpu.async_copy(table_hbm.at[idx_v], rows_v, sem).wait()
        pltpu.sync_copy(rows_v, out_hbm.at[pl.ds(base, b_per_w)])

    return sc_gather


def _proj_t_body(wt_ref, xt_ref, b_ref, o_ref):
    acc = lax.dot_general(
        wt_ref[...],
        xt_ref[...],
        dimension_numbers=(((0,), (0,)), ((), ())),
        preferred_element_type=jnp.float32,
    )
    o_ref[...] = acc + jnp.transpose(b_ref[...], (1, 0))


def _tc_project_t(xt, WT, brow, vblk):
    D, B = xt.shape
    V = WT.shape[1]
    return pl.pallas_call(
        _proj_t_body,
        grid=(pl.cdiv(V, vblk),),
        in_specs=[
            pl.BlockSpec((D, vblk), lambda i: (0, i)),
            pl.BlockSpec((D, B), lambda i: (0, 0)),
            pl.BlockSpec((1, vblk), lambda i: (0, i)),
        ],
        out_specs=pl.BlockSpec((vblk, B), lambda i: (i, 0)),
        out_shape=jax.ShapeDtypeStruct((V, B), jnp.float32),
    )(WT, xt, brow)


def kernel(inputs, table, W, b):
    V, D = table.shape
    B = inputs.shape[0]
    idx = inputs.astype(jnp.int32)
    x = _make_sc_gather(V, D, B)(table, idx)
    logits_t = _tc_project_t(x.T, W.T, b.reshape(1, V), 2048)
    return logits_t.T


# speedup vs baseline: 2.7450x; 1.0031x over previous
import functools

import jax
import jax.numpy as jnp
from jax import lax
from jax.experimental import pallas as pl
from jax.experimental.pallas import tpu as pltpu
from jax.experimental.pallas import tpu_sc as plsc


def _make_sc_gather(V, DP, B):
    info = plsc.get_sparse_core_info()
    NC, NS = info.num_cores, info.num_subcores
    NW = NC * NS
    b_per_w = B // NW
    mesh = plsc.VectorSubcoreMesh(core_axis_name="c", subcore_axis_name="s")

    @functools.partial(
        pl.kernel,
        mesh=mesh,
        out_type=jax.ShapeDtypeStruct((B, DP), jnp.float32),
        scratch_types=[
            pltpu.VMEM((b_per_w,), jnp.int32),
            pltpu.VMEM((b_per_w, DP), jnp.float32),
            pltpu.SemaphoreType.DMA,
        ],
    )
    def sc_gather(table_hbm, idx_hbm, out_hbm, idx_v, rows_v, sem):
        wid = lax.axis_index("s") * NC + lax.axis_index("c")
        base = wid * b_per_w
        pltpu.sync_copy(idx_hbm.at[pl.ds(base, b_per_w)], idx_v)
        pltpu.async_copy(table_hbm.at[idx_v], rows_v, sem).wait()
        pltpu.sync_copy(rows_v, out_hbm.at[pl.ds(base, b_per_w)])

    return sc_gather


def _proj_t_body(wt_ref, xt_ref, b_ref, o_ref):
    acc = lax.dot_general(
        wt_ref[...],
        xt_ref[...],
        dimension_numbers=(((0,), (0,)), ((), ())),
        preferred_element_type=jnp.float32,
    )
    o_ref[...] = acc + jnp.transpose(b_ref[...], (1, 0))


def _tc_project_t(xt, WT, brow, vblk):
    D, B = xt.shape
    V = WT.shape[1]
    return pl.pallas_call(
        _proj_t_body,
        grid=(pl.cdiv(V, vblk),),
        in_specs=[
            pl.BlockSpec((D, vblk), lambda i: (0, i)),
            pl.BlockSpec((D, B), lambda i: (0, 0)),
            pl.BlockSpec((1, vblk), lambda i: (0, i)),
        ],
        out_specs=pl.BlockSpec((vblk, B), lambda i: (i, 0)),
        out_shape=jax.ShapeDtypeStruct((V, B), jnp.float32),
    )(WT, xt, brow)


def kernel(inputs, table, W, b):
    V, D = table.shape
    B = inputs.shape[0]
    idx = inputs.astype(jnp.int32)
    table_p = jnp.pad(table, ((0, 0), (0, 128 - D)))
    xp = _make_sc_gather(V, 128, B)(table_p, idx)
    logits_t = _tc_project_t(xp[:, :D].T, W.T, b.reshape(1, V), 2048)
    return logits_t.T


# speedup vs baseline: 2.8262x; 1.0296x over previous
import functools

import jax
import jax.numpy as jnp
from jax import lax
from jax.experimental import pallas as pl
from jax.experimental.pallas import tpu as pltpu
from jax.experimental.pallas import tpu_sc as plsc


def _make_sc_gather(V, DP, B):
    info = plsc.get_sparse_core_info()
    NC, NS = info.num_cores, info.num_subcores
    NW = NC * NS
    b_per_w = B // NW
    mesh = plsc.VectorSubcoreMesh(core_axis_name="c", subcore_axis_name="s")

    @functools.partial(
        pl.kernel,
        mesh=mesh,
        out_type=jax.ShapeDtypeStruct((B, DP), jnp.float32),
        scratch_types=[
            pltpu.VMEM((b_per_w,), jnp.int32),
            pltpu.VMEM((b_per_w, DP), jnp.float32),
            pltpu.SemaphoreType.DMA,
        ],
    )
    def sc_gather(table_hbm, idx_hbm, out_hbm, idx_v, rows_v, sem):
        wid = lax.axis_index("s") * NC + lax.axis_index("c")
        base = wid * b_per_w
        pltpu.sync_copy(idx_hbm.at[pl.ds(base, b_per_w)], idx_v)
        pltpu.async_copy(table_hbm.at[idx_v], rows_v, sem).wait()
        pltpu.sync_copy(rows_v, out_hbm.at[pl.ds(base, b_per_w)])

    return sc_gather


def _padT_body(tt_ref, o_ref):
    o_ref[:, : tt_ref.shape[0]] = jnp.transpose(tt_ref[...], (1, 0))


def _pad_transpose(tableT, vblk):
    D, V = tableT.shape
    return pl.pallas_call(
        _padT_body,
        grid=(pl.cdiv(V, vblk),),
        in_specs=[pl.BlockSpec((D, vblk), lambda i: (0, i))],
        out_specs=pl.BlockSpec((vblk, 128), lambda i: (i, 0)),
        out_shape=jax.ShapeDtypeStruct((V, 128), jnp.float32),
    )(tableT)


def _proj_t_body(wt_ref, xt_ref, b_ref, o_ref):
    acc = lax.dot_general(
        wt_ref[...],
        xt_ref[...],
        dimension_numbers=(((0,), (0,)), ((), ())),
        preferred_element_type=jnp.float32,
    )
    o_ref[...] = acc + jnp.transpose(b_ref[...], (1, 0))


def _tc_project_t(xt, WT, brow, vblk):
    D, B = xt.shape
    V = WT.shape[1]
    return pl.pallas_call(
        _proj_t_body,
        grid=(pl.cdiv(V, vblk),),
        in_specs=[
            pl.BlockSpec((D, vblk), lambda i: (0, i)),
            pl.BlockSpec((D, B), lambda i: (0, 0)),
            pl.BlockSpec((1, vblk), lambda i: (0, i)),
        ],
        out_specs=pl.BlockSpec((vblk, B), lambda i: (i, 0)),
        out_shape=jax.ShapeDtypeStruct((V, B), jnp.float32),
    )(WT, xt, brow)


def kernel(inputs, table, W, b):
    V, D = table.shape
    B = inputs.shape[0]
    idx = inputs.astype(jnp.int32)
    table_p = _pad_transpose(table.T, 2048)
    xp = _make_sc_gather(V, 128, B)(table_p, idx)
    logits_t = _tc_project_t(xp[:, :D].T, W.T, b.reshape(1, V), 2048)
    return logits_t.T


# speedup vs baseline: 2.8271x; 1.0003x over previous
import functools

import jax
import jax.numpy as jnp
from jax import lax
from jax.experimental import pallas as pl
from jax.experimental.pallas import tpu as pltpu
from jax.experimental.pallas import tpu_sc as plsc


def _make_sc_gather(V, DP, B):
    info = plsc.get_sparse_core_info()
    NC, NS = info.num_cores, info.num_subcores
    NW = NC * NS
    b_per_w = B // NW
    mesh = plsc.VectorSubcoreMesh(core_axis_name="c", subcore_axis_name="s")

    @functools.partial(
        pl.kernel,
        mesh=mesh,
        out_type=jax.ShapeDtypeStruct((B, DP), jnp.float32),
        scratch_types=[
            pltpu.VMEM((b_per_w,), jnp.int32),
            pltpu.VMEM((b_per_w, DP), jnp.float32),
            pltpu.SemaphoreType.DMA,
        ],
    )
    def sc_gather(table_hbm, idx_hbm, out_hbm, idx_v, rows_v, sem):
        wid = lax.axis_index("s") * NC + lax.axis_index("c")
        base = wid * b_per_w
        pltpu.sync_copy(idx_hbm.at[pl.ds(base, b_per_w)], idx_v)
        pltpu.async_copy(table_hbm.at[idx_v], rows_v, sem).wait()
        pltpu.sync_copy(rows_v, out_hbm.at[pl.ds(base, b_per_w)])

    return sc_gather


def _padT_body(tt_ref, o_ref):
    o_ref[:, : tt_ref.shape[0]] = jnp.transpose(tt_ref[...], (1, 0))


def _pad_transpose(tableT, vblk):
    D, V = tableT.shape
    return pl.pallas_call(
        _padT_body,
        grid=(pl.cdiv(V, vblk),),
        in_specs=[pl.BlockSpec((D, vblk), lambda i: (0, i))],
        out_specs=pl.BlockSpec((vblk, 128), lambda i: (i, 0)),
        out_shape=jax.ShapeDtypeStruct((V, 128), jnp.float32),
    )(tableT)


def _proj_t_body(wt_ref, xt_ref, b_ref, o_ref):
    acc = lax.dot_general(
        wt_ref[...],
        xt_ref[...],
        dimension_numbers=(((0,), (0,)), ((), ())),
        preferred_element_type=jnp.float32,
    )
    o_ref[...] = acc + jnp.transpose(b_ref[...], (1, 0))


def _tc_project_t(xt, WT, brow, vblk):
    D, B = xt.shape
    V = WT.shape[1]
    return pl.pallas_call(
        _proj_t_body,
        grid=(pl.cdiv(V, vblk),),
        in_specs=[
            pl.BlockSpec((D, vblk), lambda i: (0, i)),
            pl.BlockSpec((D, B), lambda i: (0, 0)),
            pl.BlockSpec((1, vblk), lambda i: (0, i)),
        ],
        out_specs=pl.BlockSpec((vblk, B), lambda i: (i, 0)),
        out_shape=jax.ShapeDtypeStruct((V, B), jnp.float32),
    )(WT, xt, brow)


def kernel(inputs, table, W, b):
    V, D = table.shape
    B = inputs.shape[0]
    idx = inputs.astype(jnp.int32)
    table_p = _pad_transpose(table.T, 2048)
    xp = _make_sc_gather(V, 128, B)(table_p, idx)
    logits_t = _tc_project_t(xp[:, :D].T, W.T, b.reshape(1, V), 4096)
    return logits_t.T
